# baseline (speedup 1.0000x reference)
import functools

import numpy as np
import jax
import jax.numpy as jnp
from jax import lax
from jax.experimental import pallas as pl
from jax.experimental.pallas import tpu as pltpu
from jax.experimental.pallas import tpu_sc as plsc

B, S, C, D = 1024, 20, 16, 128
NT = 4
R = B * S
NC, NS = 2, 16
NW = NC * NS
ROWS_PER_TILE = R // NW
CHUNK = 4
GATHER = CHUNK * C
NCHUNK = ROWS_PER_TILE // CHUNK
IDX_PER_TILE = ROWS_PER_TILE * C
BPAD = 8
ACC = NT * BPAD

V0 = 5000
BM, BN = 2048, 512
NPAD = ((V0 + BN - 1) // BN) * BN


def _sc_gather_pool(i0, i1, i2, i3, pidx, t0, t1, t2, t3):
    mesh = plsc.VectorSubcoreMesh(
        core_axis_name="c", subcore_axis_name="s",
        num_cores=NC, num_subcores=NS)

    @functools.partial(
        pl.kernel,
        out_type=jax.ShapeDtypeStruct((R * NT, D), jnp.float32),
        mesh=mesh,
        scratch_types=[
            pltpu.VMEM((NT * IDX_PER_TILE,), jnp.int32),
            pltpu.VMEM((2 * NT, GATHER), jnp.int32),
            pltpu.VMEM((NT, GATHER, D), jnp.float32),
            pltpu.VMEM((NT, GATHER, D), jnp.float32),
            pltpu.VMEM_SHARED((NS * 2 * ACC, D), jnp.float32),
            pltpu.VMEM((ACC, D), jnp.float32),
            pltpu.SemaphoreType.DMA,
            pltpu.SemaphoreType.DMA,
            pltpu.SemaphoreType.DMA,
            pltpu.SemaphoreType.DMA,
            pltpu.SemaphoreType.DMA,
            pltpu.SemaphoreType.DMA,
            pltpu.SemaphoreType.DMA,
            pltpu.SemaphoreType.DMA,
        ],
        compiler_params=pltpu.CompilerParams(use_tc_tiling_on_sc=True),
    )
    def k(ih0, ih1, ih2, ih3, pidx_hbm, tb0, tb1, tb2, tb3, out_hbm,
          idx_v, pidx_v, rows0, rows1, acc_v, z_v,
          sem_g0, sem_g1, sem_a0, sem_a1, sem_o0, sem_o1, sem_z0, sem_z1):
        sid = lax.axis_index("s")
        wid = sid * NC + lax.axis_index("c")
        base = wid * ROWS_PER_TILE
        abase = sid * 2 * ACC
        for t, ih in enumerate((ih0, ih1, ih2, ih3)):
            pltpu.sync_copy(ih.at[wid],
                            idx_v.at[pl.ds(t * IDX_PER_TILE, IDX_PER_TILE)])
        pltpu.sync_copy(pidx_hbm, pidx_v)

        @pl.loop(0, 2 * NT)
        def _(t):
            @pl.loop(0, GATHER, step=16)
            def _(g):
                pidx_v[t, pl.ds(g, 16)] = pidx_v[t, pl.ds(g, 16)] + abase

        @pl.loop(0, ACC)
        def _(r):
            @pl.loop(0, D, step=16)
            def _(c):
                z_v[r, pl.ds(c, 16)] = jnp.zeros((16,), jnp.float32)

        tables = (tb0, tb1, tb2, tb3)
        rows = (rows0, rows1)
        sem_g = (sem_g0, sem_g1)
        sem_a = (sem_a0, sem_a1)
        sem_o = (sem_o0, sem_o1)
        sem_z = (sem_z0, sem_z1)

        def gather_src(ch, t):
            return tables[t].at[
                idx_v.at[pl.ds(t * IDX_PER_TILE + ch * GATHER, GATHER)]]

        def acc_band(slab, t):
            return acc_v.at[pl.ds(abase + slab * ACC + t * BPAD, CHUNK)]

        def out_rows(ch, t):
            return out_hbm.at[pl.ds(t * R + base + ch * CHUNK, CHUNK)]

        def start_gathers(ch, slab):
            for t in range(NT):
                pltpu.async_copy(gather_src(ch, t), rows[slab].at[t],
                                 sem_g[slab])

        def wait_outs(ch, slab):
            for t in range(NT):
                pltpu.make_async_copy(acc_band(slab, t), out_rows(ch, t),
                                      sem_o[slab]).wait()

        def process(ch, slab, first):
            pltpu.async_copy(
                z_v, acc_v.at[pl.ds(abase + slab * ACC, ACC)],
                sem_z[slab]).wait()
            for t in range(NT):
                pltpu.make_async_copy(gather_src(ch, t), rows[slab].at[t],
                                      sem_g[slab]).wait()
            adds = []
            for t in range(NT):
                adds.append(pltpu.async_copy(
                    rows[slab].at[t], acc_v.at[pidx_v.at[slab * NT + t]],
                    sem_a[slab], add=True))
            for a in adds:
                a.wait()
            @pl.when(ch + 1 < NCHUNK)
            def _():
                start_gathers(ch + 1, 1 - slab)
            for t in range(NT):
                pltpu.sync_copy(acc_band(slab, t), out_rows(ch, t))

        start_gathers(0, 0)

        @pl.loop(0, NCHUNK, step=2)
        def _(ch):
            process(ch, 0, ch == 0)
            process(ch + 1, 1, ch == 0)


    return k(i0, i1, i2, i3, pidx, t0, t1, t2, t3)


def _mlp_body(v_ref, w1_ref, b1_ref, w2_ref, b2_ref, o_ref, h_ref):
    n = pl.program_id(1)

    @pl.when(n == 0)
    def _():
        acc = jnp.zeros((BM, 2 * D), jnp.float32)
        for t in range(NT):
            acc += jnp.dot(v_ref[t].astype(jnp.bfloat16), w1_ref[t],
                           preferred_element_type=jnp.float32)
        h_ref[...] = jnp.maximum(acc + b1_ref[...], 0.0).astype(jnp.bfloat16)

    o_ref[...] = (jnp.dot(h_ref[...], w2_ref[...],
                          preferred_element_type=jnp.float32)
                  + b2_ref[...])


def _mlp(v, w1, b1, w2, b2):
    grid = (R // BM, NPAD // BN)
    return pl.pallas_call(
        _mlp_body,
        grid=grid,
        in_specs=[
            pl.BlockSpec((NT, BM, D), lambda m, n: (0, m, 0)),
            pl.BlockSpec((NT, D, 2 * D), lambda m, n: (0, 0, 0)),
            pl.BlockSpec((2 * D,), lambda m, n: (0,)),
            pl.BlockSpec((2 * D, BN), lambda m, n: (0, n)),
            pl.BlockSpec((BN,), lambda m, n: (n,)),
        ],
        out_specs=pl.BlockSpec((BM, BN), lambda m, n: (m, n)),
        out_shape=jax.ShapeDtypeStruct((R, V0), jnp.float32),
        scratch_shapes=[pltpu.VMEM((BM, 2 * D), jnp.bfloat16)],
        compiler_params=pltpu.CompilerParams(
            dimension_semantics=("arbitrary", "arbitrary")),
    )(v, w1, b1, w2, b2)


_PIDX = np.asarray(
    [[s * ACC + t * BPAD + g // C for g in range(GATHER)]
     for s in range(2) for t in range(NT)],
    dtype=np.int32)


def kernel(diag_seq, drug_seq, lab_seq, proc_seq, diag_table, drug_table,
           lab_table, proc_table, W1, b1, W2, b2):
    seqs = [a.astype(jnp.int32).reshape(NW, IDX_PER_TILE)
            for a in (diag_seq, drug_seq, lab_seq, proc_seq)]
    pidx = jnp.asarray(_PIDX)

    v = _sc_gather_pool(*seqs, pidx, diag_table, drug_table, lab_table,
                        proc_table)
    v = v.reshape(NT, R, D)

    w1 = W1.reshape(NT, D, 2 * D).astype(jnp.bfloat16)
    w2 = jnp.pad(W2, ((0, 0), (0, NPAD - V0))).astype(jnp.bfloat16)
    b2p = jnp.pad(b2, (0, NPAD - V0))

    out = _mlp(v, w1, b1, w2, b2p)
    return out.reshape(B, S, V0)


# speedup vs baseline: 1.6885x; 1.6885x over previous
import functools

import numpy as np
import jax
import jax.numpy as jnp
from jax import lax
from jax.experimental import pallas as pl
from jax.experimental.pallas import tpu as pltpu
from jax.experimental.pallas import tpu_sc as plsc

B, S, C, D = 1024, 20, 16, 128
NT = 4
R = B * S
NC, NS = 2, 16
NW = NC * NS
ROWS_PER_TILE = R // NW
CHUNK = 4
GATHER = CHUNK * C
NCHUNK = ROWS_PER_TILE // CHUNK
IDX_PER_TILE = ROWS_PER_TILE * C
BPAD = 8
ACC = NT * BPAD

V0 = 5000
BM, BN = 2048, 512
NPAD = ((V0 + BN - 1) // BN) * BN


def _sc_gather_pool(i0, i1, i2, i3, pidx, t0, t1, t2, t3):
    mesh = plsc.VectorSubcoreMesh(
        core_axis_name="c", subcore_axis_name="s",
        num_cores=NC, num_subcores=NS)

    @functools.partial(
        pl.kernel,
        out_type=jax.ShapeDtypeStruct((R * NT, D), jnp.float32),
        mesh=mesh,
        scratch_types=[
            pltpu.VMEM((NT * IDX_PER_TILE,), jnp.int32),
            pltpu.VMEM((2 * NT, GATHER), jnp.int32),
            pltpu.VMEM((NT, GATHER, D), jnp.float32),
            pltpu.VMEM((NT, GATHER, D), jnp.float32),
            pltpu.VMEM_SHARED((NS * 2 * ACC, D), jnp.float32),
            pltpu.VMEM((ACC, D), jnp.float32),
            pltpu.SemaphoreType.DMA,
            pltpu.SemaphoreType.DMA,
            pltpu.SemaphoreType.DMA,
            pltpu.SemaphoreType.DMA,
            pltpu.SemaphoreType.DMA,
            pltpu.SemaphoreType.DMA,
            pltpu.SemaphoreType.DMA,
            pltpu.SemaphoreType.DMA,
        ],
        compiler_params=pltpu.CompilerParams(use_tc_tiling_on_sc=True),
    )
    def k(ih0, ih1, ih2, ih3, pidx_hbm, tb0, tb1, tb2, tb3, out_hbm,
          idx_v, pidx_v, rows0, rows1, acc_v, z_v,
          sem_g0, sem_g1, sem_a0, sem_a1, sem_o0, sem_o1, sem_z0, sem_z1):
        sid = lax.axis_index("s")
        wid = sid * NC + lax.axis_index("c")
        base = wid * ROWS_PER_TILE
        abase = sid * 2 * ACC
        for t, ih in enumerate((ih0, ih1, ih2, ih3)):
            pltpu.sync_copy(ih.at[wid],
                            idx_v.at[pl.ds(t * IDX_PER_TILE, IDX_PER_TILE)])
        pltpu.sync_copy(pidx_hbm, pidx_v)

        @pl.loop(0, 2 * NT)
        def _(t):
            @pl.loop(0, GATHER, step=16)
            def _(g):
                pidx_v[t, pl.ds(g, 16)] = pidx_v[t, pl.ds(g, 16)] + abase

        @pl.loop(0, ACC)
        def _(r):
            @pl.loop(0, D, step=16)
            def _(c):
                z_v[r, pl.ds(c, 16)] = jnp.zeros((16,), jnp.float32)

        tables = (tb0, tb1, tb2, tb3)
        rows = (rows0, rows1)
        sem_g = (sem_g0, sem_g1)
        sem_a = (sem_a0, sem_a1)
        sem_o = (sem_o0, sem_o1)
        sem_z = (sem_z0, sem_z1)

        def gather_src(ch, t):
            return tables[t].at[
                idx_v.at[pl.ds(t * IDX_PER_TILE + ch * GATHER, GATHER)]]

        def acc_band(slab, t):
            return acc_v.at[pl.ds(abase + slab * ACC + t * BPAD, CHUNK)]

        def out_rows(ch, t):
            return out_hbm.at[pl.ds(t * R + base + ch * CHUNK, CHUNK)]

        def start_gathers(ch, slab):
            for t in range(NT):
                pltpu.async_copy(gather_src(ch, t), rows[slab].at[t],
                                 sem_g[slab])

        def wait_outs(ch, slab):
            for t in range(NT):
                pltpu.make_async_copy(acc_band(slab, t), out_rows(ch, t),
                                      sem_o[slab]).wait()

        def process(ch, slab, first):
            pltpu.async_copy(
                z_v, acc_v.at[pl.ds(abase + slab * ACC, ACC)],
                sem_z[slab]).wait()
            for t in range(NT):
                pltpu.make_async_copy(gather_src(ch, t), rows[slab].at[t],
                                      sem_g[slab]).wait()
            adds = []
            for t in range(NT):
                adds.append(pltpu.async_copy(
                    rows[slab].at[t], acc_v.at[pidx_v.at[slab * NT + t]],
                    sem_a[slab], add=True))
            for a in adds:
                a.wait()
            @pl.when(ch + 1 < NCHUNK)
            def _():
                start_gathers(ch + 1, 1 - slab)
            for t in range(NT):
                pltpu.sync_copy(acc_band(slab, t), out_rows(ch, t))

        start_gathers(0, 0)

        @pl.loop(0, NCHUNK, step=2)
        def _(ch):
            process(ch, 0, ch == 0)
            process(ch + 1, 1, ch == 0)


    return k(i0, i1, i2, i3, pidx, t0, t1, t2, t3)


def _mlp_body(v_ref, w1_ref, b1_ref, w2_ref, b2_ref, o_ref, h_ref):
    n = pl.program_id(1)

    @pl.when(n == 0)
    def _():
        acc = jnp.zeros((BM, 2 * D), jnp.float32)
        for t in range(NT):
            acc += jnp.dot(v_ref[t].astype(jnp.bfloat16), w1_ref[t],
                           preferred_element_type=jnp.float32)
        h = jnp.maximum(acc + b1_ref[...], 0.0).astype(jnp.bfloat16)
        h_ref[...] = h.T

    for si in range(BM // B):
        o_ref[si] = (jnp.dot(w2_ref[...], h_ref[:, pl.ds(si * B, B)],
                             preferred_element_type=jnp.float32)
                     + b2_ref[...][:, None])


def _mlp(v, w1, b1, w2, b2):
    grid = (R // BM, NPAD // BN)
    return pl.pallas_call(
        _mlp_body,
        grid=grid,
        in_specs=[
            pl.BlockSpec((NT, BM, D), lambda m, n: (0, m, 0)),
            pl.BlockSpec((NT, D, 2 * D), lambda m, n: (0, 0, 0)),
            pl.BlockSpec((2 * D,), lambda m, n: (0,)),
            pl.BlockSpec((BN, 2 * D), lambda m, n: (n, 0)),
            pl.BlockSpec((BN,), lambda m, n: (n,)),
        ],
        out_specs=pl.BlockSpec((BM // B, BN, B), lambda m, n: (m, n, 0)),
        out_shape=jax.ShapeDtypeStruct((S, V0, B), jnp.float32),
        scratch_shapes=[pltpu.VMEM((2 * D, BM), jnp.bfloat16)],
        compiler_params=pltpu.CompilerParams(
            dimension_semantics=("arbitrary", "arbitrary")),
    )(v, w1, b1, w2, b2)


_PIDX = np.asarray(
    [[s * ACC + t * BPAD + g // C for g in range(GATHER)]
     for s in range(2) for t in range(NT)],
    dtype=np.int32)


def kernel(diag_seq, drug_seq, lab_seq, proc_seq, diag_table, drug_table,
           lab_table, proc_table, W1, b1, W2, b2):
    seqs = [a.astype(jnp.int32).transpose(1, 0, 2).reshape(NW, IDX_PER_TILE)
            for a in (diag_seq, drug_seq, lab_seq, proc_seq)]
    pidx = jnp.asarray(_PIDX)

    v = _sc_gather_pool(*seqs, pidx, diag_table, drug_table, lab_table,
                        proc_table)
    v = v.reshape(NT, R, D)

    w1 = W1.reshape(NT, D, 2 * D).astype(jnp.bfloat16)
    w2t = jnp.pad(W2, ((0, 0), (0, NPAD - V0))).T.astype(jnp.bfloat16)
    b2p = jnp.pad(b2, (0, NPAD - V0))

    out = _mlp(v, w1, b1, w2t, b2p)
    return out.transpose(2, 0, 1)


# speedup vs baseline: 1.6890x; 1.0003x over previous
import functools

import numpy as np
import jax
import jax.numpy as jnp
from jax import lax
from jax.experimental import pallas as pl
from jax.experimental.pallas import tpu as pltpu
from jax.experimental.pallas import tpu_sc as plsc

B, S, C, D = 1024, 20, 16, 128
NT = 4
R = B * S
NC, NS = 2, 16
NW = NC * NS
ROWS_PER_TILE = R // NW
CHUNK = 4
GATHER = CHUNK * C
NCHUNK = ROWS_PER_TILE // CHUNK
IDX_PER_TILE = ROWS_PER_TILE * C
BPAD = 8
ACC = NT * BPAD

V0 = 5000
NSLICE = 1
BM, BN = 2048, 512
NPAD = ((V0 + BN - 1) // BN) * BN


def _sc_gather_pool(i0, i1, i2, i3, pidx, t0, t1, t2, t3, rows_total):
    rows_per_tile = rows_total // NW
    nchunk = rows_per_tile // CHUNK
    idx_per_tile = rows_per_tile * C
    mesh = plsc.VectorSubcoreMesh(
        core_axis_name="c", subcore_axis_name="s",
        num_cores=NC, num_subcores=NS)

    @functools.partial(
        pl.kernel,
        out_type=jax.ShapeDtypeStruct((rows_total * NT, D), jnp.float32),
        mesh=mesh,
        scratch_types=[
            pltpu.VMEM((NT * idx_per_tile,), jnp.int32),
            pltpu.VMEM((2 * NT, GATHER), jnp.int32),
            pltpu.VMEM((NT, GATHER, D), jnp.float32),
            pltpu.VMEM((NT, GATHER, D), jnp.float32),
            pltpu.VMEM_SHARED((NS * 2 * ACC, D), jnp.float32),
            pltpu.VMEM((ACC, D), jnp.float32),
            pltpu.SemaphoreType.DMA,
            pltpu.SemaphoreType.DMA,
            pltpu.SemaphoreType.DMA,
            pltpu.SemaphoreType.DMA,
            pltpu.SemaphoreType.DMA,
            pltpu.SemaphoreType.DMA,
            pltpu.SemaphoreType.DMA,
            pltpu.SemaphoreType.DMA,
        ],
        compiler_params=pltpu.CompilerParams(use_tc_tiling_on_sc=True),
    )
    def k(ih0, ih1, ih2, ih3, pidx_hbm, tb0, tb1, tb2, tb3, out_hbm,
          idx_v, pidx_v, rows0, rows1, acc_v, z_v,
          sem_g0, sem_g1, sem_a0, sem_a1, sem_o0, sem_o1, sem_z0, sem_z1):
        sid = lax.axis_index("s")
        wid = sid * NC + lax.axis_index("c")
        base = wid * rows_per_tile
        abase = sid * 2 * ACC
        for t, ih in enumerate((ih0, ih1, ih2, ih3)):
            pltpu.sync_copy(ih.at[wid],
                            idx_v.at[pl.ds(t * idx_per_tile, idx_per_tile)])
        pltpu.sync_copy(pidx_hbm, pidx_v)

        @pl.loop(0, 2 * NT)
        def _(t):
            @pl.loop(0, GATHER, step=16)
            def _(g):
                pidx_v[t, pl.ds(g, 16)] = pidx_v[t, pl.ds(g, 16)] + abase

        @pl.loop(0, ACC)
        def _(r):
            @pl.loop(0, D, step=16)
            def _(c):
                z_v[r, pl.ds(c, 16)] = jnp.zeros((16,), jnp.float32)

        tables = (tb0, tb1, tb2, tb3)
        rows = (rows0, rows1)
        sem_g = (sem_g0, sem_g1)
        sem_a = (sem_a0, sem_a1)
        sem_o = (sem_o0, sem_o1)
        sem_z = (sem_z0, sem_z1)

        def gather_src(ch, t):
            return tables[t].at[
                idx_v.at[pl.ds(t * idx_per_tile + ch * GATHER, GATHER)]]

        def acc_band(slab, t):
            return acc_v.at[pl.ds(abase + slab * ACC + t * BPAD, CHUNK)]

        def out_rows(ch, t):
            return out_hbm.at[pl.ds(t * rows_total + base + ch * CHUNK,
                                    CHUNK)]

        def start_gathers(ch, slab):
            for t in range(NT):
                pltpu.async_copy(gather_src(ch, t), rows[slab].at[t],
                                 sem_g[slab])

        def wait_outs(ch, slab):
            for t in range(NT):
                pltpu.make_async_copy(acc_band(slab, t), out_rows(ch, t),
                                      sem_o[slab]).wait()

        def process(ch, slab, first):
            pltpu.async_copy(
                z_v, acc_v.at[pl.ds(abase + slab * ACC, ACC)],
                sem_z[slab]).wait()
            for t in range(NT):
                pltpu.make_async_copy(gather_src(ch, t), rows[slab].at[t],
                                      sem_g[slab]).wait()
            adds = []
            for t in range(NT):
                adds.append(pltpu.async_copy(
                    rows[slab].at[t], acc_v.at[pidx_v.at[slab * NT + t]],
                    sem_a[slab], add=True))
            for a in adds:
                a.wait()
            @pl.when(ch + 1 < nchunk)
            def _():
                start_gathers(ch + 1, 1 - slab)
            for t in range(NT):
                pltpu.sync_copy(acc_band(slab, t), out_rows(ch, t))

        start_gathers(0, 0)

        @pl.loop(0, NCHUNK, step=2)
        def _(ch):
            process(ch, 0, ch == 0)
            process(ch + 1, 1, ch == 0)


    return k(i0, i1, i2, i3, pidx, t0, t1, t2, t3)


def _mlp_body(v_ref, w1_ref, b1_ref, w2_ref, b2_ref, o_ref, h_ref):
    n = pl.program_id(1)

    @pl.when(n == 0)
    def _():
        acc = jnp.zeros((BM, 2 * D), jnp.float32)
        for t in range(NT):
            acc += jnp.dot(v_ref[t].astype(jnp.bfloat16), w1_ref[t],
                           preferred_element_type=jnp.float32)
        h = jnp.maximum(acc + b1_ref[...], 0.0).astype(jnp.bfloat16)
        h_ref[...] = h.T

    for si in range(BM // B):
        o_ref[si] = (jnp.dot(w2_ref[...], h_ref[:, pl.ds(si * B, B)],
                             preferred_element_type=jnp.float32)
                     + b2_ref[...][:, None])


def _mlp(v, w1, b1, w2, b2, s_off, prev=None):
    s_count = S // NSLICE
    grid = (s_count * B // BM, NPAD // BN)
    mb = s_off // (BM // B)
    in_specs = [
        pl.BlockSpec((NT, BM, D), lambda m, n: (0, m, 0)),
        pl.BlockSpec((NT, D, 2 * D), lambda m, n: (0, 0, 0)),
        pl.BlockSpec((2 * D,), lambda m, n: (0,)),
        pl.BlockSpec((BN, 2 * D), lambda m, n: (n, 0)),
        pl.BlockSpec((BN,), lambda m, n: (n,)),
    ]
    args = [v, w1, b1, w2, b2]
    aliases = {}
    if prev is not None:
        in_specs.append(pl.BlockSpec(memory_space=pl.ANY))
        args.append(prev)
        aliases = {5: 0}

    def body(*refs):
        _mlp_body(*refs[:5], refs[-2], refs[-1])

    return pl.pallas_call(
        body,
        grid=grid,
        in_specs=in_specs,
        out_specs=pl.BlockSpec((BM // B, BN, B),
                               lambda m, n: (m + mb, n, 0)),
        out_shape=jax.ShapeDtypeStruct((S, V0, B), jnp.float32),
        scratch_shapes=[pltpu.VMEM((2 * D, BM), jnp.bfloat16)],
        input_output_aliases=aliases,
        compiler_params=pltpu.CompilerParams(
            dimension_semantics=("arbitrary", "arbitrary")),
    )(*args)


_PIDX = np.asarray(
    [[s * ACC + t * BPAD + g // C for g in range(GATHER)]
     for s in range(2) for t in range(NT)],
    dtype=np.int32)


def kernel(diag_seq, drug_seq, lab_seq, proc_seq, diag_table, drug_table,
           lab_table, proc_table, W1, b1, W2, b2):
    seqs_t = [a.astype(jnp.int32).transpose(1, 0, 2)
              for a in (diag_seq, drug_seq, lab_seq, proc_seq)]
    pidx = jnp.asarray(_PIDX)

    w1 = W1.reshape(NT, D, 2 * D).astype(jnp.bfloat16)
    w2t = jnp.pad(W2, ((0, 0), (0, NPAD - V0))).T.astype(jnp.bfloat16)
    b2p = jnp.pad(b2, (0, NPAD - V0))

    s_slice = S // NSLICE
    rows_slice = s_slice * B
    outs = []
    vs = [
        _sc_gather_pool(
            *[st[j * s_slice:(j + 1) * s_slice]
              .reshape(NW, rows_slice * C // NW) for st in seqs_t],
            pidx, diag_table, drug_table, lab_table, proc_table,
            rows_slice)
        for j in range(NSLICE)
    ]
    out = None
    for j in range(NSLICE):
        v = vs[j].reshape(NT, rows_slice, D)
        out = _mlp(v, w1, b1, w2t, b2p, j * s_slice, prev=out)
    return out.transpose(2, 0, 1)


# speedup vs baseline: 1.8843x; 1.1156x over previous
import functools

import numpy as np
import jax
import jax.numpy as jnp
from jax import lax
from jax.experimental import pallas as pl
from jax.experimental.pallas import tpu as pltpu
from jax.experimental.pallas import tpu_sc as plsc

B, S, C, D = 1024, 20, 16, 128
NT = 4
R = B * S
NC, NS = 2, 16
NW = NC * NS
ROWS_PER_TILE = R // NW
CHUNK = 8
GATHER = CHUNK * C
IDX_PER_TILE = ROWS_PER_TILE * C
BPAD = 8
ACC = NT * BPAD

V0 = 5000
NSLICE = 1
BM, BN = 2048, 512
NPAD = ((V0 + BN - 1) // BN) * BN


def _sc_gather_pool(i0, i1, i2, i3, pidx, t0, t1, t2, t3, rows_total):
    rows_per_tile = rows_total // NW
    nchunk = rows_per_tile // CHUNK
    idx_per_tile = rows_per_tile * C
    mesh = plsc.VectorSubcoreMesh(
        core_axis_name="c", subcore_axis_name="s",
        num_cores=NC, num_subcores=NS)

    @functools.partial(
        pl.kernel,
        out_type=jax.ShapeDtypeStruct((rows_total * NT, D), jnp.float32),
        mesh=mesh,
        scratch_types=[
            pltpu.VMEM((NT * idx_per_tile,), jnp.int32),
            pltpu.VMEM((2 * NT, GATHER), jnp.int32),
            pltpu.VMEM((NT, GATHER, D), jnp.float32),
            pltpu.VMEM_SHARED((NS * 2 * ACC, D), jnp.float32),
            pltpu.VMEM((ACC, D), jnp.float32),
            pltpu.SemaphoreType.DMA,
            pltpu.SemaphoreType.DMA,
            pltpu.SemaphoreType.DMA,
            pltpu.SemaphoreType.DMA,
            pltpu.SemaphoreType.DMA,
            pltpu.SemaphoreType.DMA,
            pltpu.SemaphoreType.DMA,
            pltpu.SemaphoreType.DMA,
        ],
        compiler_params=pltpu.CompilerParams(use_tc_tiling_on_sc=True),
    )
    def k(ih0, ih1, ih2, ih3, pidx_hbm, tb0, tb1, tb2, tb3, out_hbm,
          idx_v, pidx_v, rows_v, acc_v, z_v,
          sem_g0, sem_g1, sem_a0, sem_a1, sem_o0, sem_o1, sem_z0, sem_z1):
        sid = lax.axis_index("s")
        wid = sid * NC + lax.axis_index("c")
        base = wid * rows_per_tile
        abase = sid * 2 * ACC
        for t, ih in enumerate((ih0, ih1, ih2, ih3)):
            pltpu.sync_copy(ih.at[wid],
                            idx_v.at[pl.ds(t * idx_per_tile, idx_per_tile)])
        pltpu.sync_copy(pidx_hbm, pidx_v)

        @pl.loop(0, 2 * NT)
        def _(t):
            @pl.loop(0, GATHER, step=16)
            def _(g):
                pidx_v[t, pl.ds(g, 16)] = pidx_v[t, pl.ds(g, 16)] + abase

        @pl.loop(0, ACC)
        def _(r):
            @pl.loop(0, D, step=16)
            def _(c):
                z_v[r, pl.ds(c, 16)] = jnp.zeros((16,), jnp.float32)

        tables = (tb0, tb1, tb2, tb3)
        sem_a = (sem_a0, sem_a1)
        sem_o = (sem_o0, sem_o1)
        sem_z = (sem_z0, sem_z1)

        def gather_src(ch, t):
            return tables[t].at[
                idx_v.at[pl.ds(t * idx_per_tile + ch * GATHER, GATHER)]]

        def acc_band(slab, t):
            return acc_v.at[pl.ds(abase + slab * ACC + t * BPAD, CHUNK)]

        def out_rows(ch, t):
            return out_hbm.at[pl.ds(t * rows_total + base + ch * CHUNK,
                                    CHUNK)]

        def start_gathers(ch):
            for t in range(NT):
                pltpu.async_copy(gather_src(ch, t), rows_v.at[t], sem_g0)

        def wait_outs(ch, slab):
            for t in range(NT):
                pltpu.make_async_copy(acc_band(slab, t), out_rows(ch, t),
                                      sem_o[slab]).wait()

        def process(ch, slab, first):
            pltpu.async_copy(
                z_v, acc_v.at[pl.ds(abase + slab * ACC, ACC)],
                sem_z[slab]).wait()
            for t in range(NT):
                pltpu.make_async_copy(gather_src(ch, t), rows_v.at[t],
                                      sem_g0).wait()
            adds = []
            for t in range(NT):
                adds.append(pltpu.async_copy(
                    rows_v.at[t], acc_v.at[pidx_v.at[slab * NT + t]],
                    sem_a[slab], add=True))
            for a in adds:
                a.wait()
            @pl.when(ch + 1 < nchunk)
            def _():
                start_gathers(ch + 1)
            for t in range(NT):
                pltpu.sync_copy(acc_band(slab, t), out_rows(ch, t))

        start_gathers(0)

        @pl.loop(0, nchunk, step=2)
        def _(ch):
            process(ch, 0, ch == 0)
            process(ch + 1, 1, ch == 0)


    return k(i0, i1, i2, i3, pidx, t0, t1, t2, t3)


def _mlp_body(v_ref, w1_ref, b1_ref, w2_ref, b2_ref, o_ref, h_ref):
    n = pl.program_id(1)

    @pl.when(n == 0)
    def _():
        acc = jnp.zeros((BM, 2 * D), jnp.float32)
        for t in range(NT):
            acc += jnp.dot(v_ref[t].astype(jnp.bfloat16), w1_ref[t],
                           preferred_element_type=jnp.float32)
        h = jnp.maximum(acc + b1_ref[...], 0.0).astype(jnp.bfloat16)
        h_ref[...] = h.T

    for si in range(BM // B):
        o_ref[si] = (jnp.dot(w2_ref[...], h_ref[:, pl.ds(si * B, B)],
                             preferred_element_type=jnp.float32)
                     + b2_ref[...][:, None])


def _mlp(v, w1, b1, w2, b2, s_off, prev=None):
    s_count = S // NSLICE
    grid = (s_count * B // BM, NPAD // BN)
    mb = s_off // (BM // B)
    in_specs = [
        pl.BlockSpec((NT, BM, D), lambda m, n: (0, m, 0)),
        pl.BlockSpec((NT, D, 2 * D), lambda m, n: (0, 0, 0)),
        pl.BlockSpec((2 * D,), lambda m, n: (0,)),
        pl.BlockSpec((BN, 2 * D), lambda m, n: (n, 0)),
        pl.BlockSpec((BN,), lambda m, n: (n,)),
    ]
    args = [v, w1, b1, w2, b2]
    aliases = {}
    if prev is not None:
        in_specs.append(pl.BlockSpec(memory_space=pl.ANY))
        args.append(prev)
        aliases = {5: 0}

    def body(*refs):
        _mlp_body(*refs[:5], refs[-2], refs[-1])

    return pl.pallas_call(
        body,
        grid=grid,
        in_specs=in_specs,
        out_specs=pl.BlockSpec((BM // B, BN, B),
                               lambda m, n: (m + mb, n, 0)),
        out_shape=jax.ShapeDtypeStruct((S, V0, B), jnp.float32),
        scratch_shapes=[pltpu.VMEM((2 * D, BM), jnp.bfloat16)],
        input_output_aliases=aliases,
        compiler_params=pltpu.CompilerParams(
            dimension_semantics=("arbitrary", "arbitrary")),
    )(*args)


_PIDX = np.asarray(
    [[s * ACC + t * BPAD + g // C for g in range(GATHER)]
     for s in range(2) for t in range(NT)],
    dtype=np.int32)


def kernel(diag_seq, drug_seq, lab_seq, proc_seq, diag_table, drug_table,
           lab_table, proc_table, W1, b1, W2, b2):
    seqs_t = [a.astype(jnp.int32).transpose(1, 0, 2)
              for a in (diag_seq, drug_seq, lab_seq, proc_seq)]
    pidx = jnp.asarray(_PIDX)

    w1 = W1.reshape(NT, D, 2 * D).astype(jnp.bfloat16)
    w2t = jnp.pad(W2, ((0, 0), (0, NPAD - V0))).T.astype(jnp.bfloat16)
    b2p = jnp.pad(b2, (0, NPAD - V0))

    s_slice = S // NSLICE
    rows_slice = s_slice * B
    outs = []
    vs = [
        _sc_gather_pool(
            *[st[j * s_slice:(j + 1) * s_slice]
              .reshape(NW, rows_slice * C // NW) for st in seqs_t],
            pidx, diag_table, drug_table, lab_table, proc_table,
            rows_slice)
        for j in range(NSLICE)
    ]
    out = None
    for j in range(NSLICE):
        v = vs[j].reshape(NT, rows_slice, D)
        out = _mlp(v, w1, b1, w2t, b2p, j * s_slice, prev=out)
    return out.transpose(2, 0, 1)
